# baseline (speedup 1.0000x reference)
import functools

import jax
import jax.numpy as jnp
from jax import lax
from jax.experimental import pallas as pl
from jax.experimental.pallas import tpu as pltpu
from jax.experimental.pallas import tpu_sc as plsc

N = 16384
D = 384
CD = 882
ROWS = 1024
EPS_LN = 1e-3


def _ln(x, g, b):
    mu = jnp.mean(x, axis=-1, keepdims=True)
    var = jnp.mean((x - mu) ** 2, axis=-1, keepdims=True)
    return (x - mu) / jnp.sqrt(var + EPS_LN) * g + b


def _row_spec(cols):
    return pl.BlockSpec((ROWS, cols), lambda i: (i, 0))


def _full_spec(shape):
    nd = len(shape)
    return pl.BlockSpec(shape, lambda i: (0,) * nd)


def _stage_call(body, outs_cols, row_args, full_args):
    grid = N // ROWS
    in_specs = [_row_spec(a.shape[-1]) for a in row_args] + [
        _full_spec(a.shape) for a in full_args
    ]
    out_specs = [_row_spec(c) for c in outs_cols]
    out_shape = [jax.ShapeDtypeStruct((N, c), jnp.float32)
                 for c in outs_cols]
    if len(outs_cols) == 1:
        out_specs, out_shape = out_specs[0], out_shape[0]
    return pl.pallas_call(
        body,
        grid=(grid,),
        in_specs=in_specs,
        out_specs=out_specs,
        out_shape=out_shape,
    )(*row_args, *full_args)


def _mm(x, w):
    return jnp.dot(x, w, preferred_element_type=jnp.float32)


def _k1_body(net, inp, corr, w1, b1, w2, b2, lng, lnb, w3, b3, ng, nb, out):
    h = jnp.maximum(_mm(corr[...], w1[...]) + b1[...], 0.0)
    h = _mm(h, w2[...]) + b2[...]
    h = jnp.maximum(_ln(h, lng[...], lnb[...]), 0.0)
    h = _mm(h, w3[...]) + b3[...]
    out[...] = _ln(net[...] + inp[...] + h, ng[...], nb[...])


def _k2_body(x, g, wa, ba, wb, bb, out):
    t = jnp.maximum(_mm(g[...], wa[...]) + ba[...], 0.0)
    out[...] = x[...] + _mm(t, wb[...]) + bb[...]


def _k23_body(x, g, wa, ba, wb, bb, gw, gb, fw, fb, no, eo, fo):
    t = jnp.maximum(_mm(g[...], wa[...]) + ba[...], 0.0)
    nv = x[...] + _mm(t, wb[...]) + bb[...]
    no[...] = nv
    eo[...] = jnp.exp(_mm(nv, gw[...]) + gb[...])
    fo[...] = _mm(nv, fw[...]) + fb[...]


def _k4_body(e, fx, den, out):
    out[...] = fx[...] * (e[...] / jnp.maximum(den[...], 1e-6))


def _k5a_body(x, y, hw, hb, gw, gb, fw, fb, no, eo, fo):
    nv = x[...] + _mm(y[...], hw[...]) + hb[...]
    no[...] = nv
    eo[...] = jnp.exp(_mm(nv, gw[...]) + gb[...])
    fo[...] = _mm(nv, fw[...]) + fb[...]


def _gated_res(x, gw, gb, r1w, r1b, r2w, r2b):
    gate = jax.nn.sigmoid(_mm(x, gw) + gb)
    r = jnp.maximum(_mm(x, r1w) + r1b, 0.0)
    r = _mm(r, r2w) + r2b
    return x * gate + r


def _k5b_body(x, y, hw, hb, ln1g, ln1b, g1w, g1b, r11w, r11b, r12w, r12b,
              ln2g, ln2b, g2w, g2b, r21w, r21b, r22w, r22b, dw, db, no, ho):
    nv = x[...] + _mm(y[...], hw[...]) + hb[...]
    nv = _ln(nv, ln1g[...], ln1b[...])
    nv = _gated_res(nv, g1w[...], g1b[...], r11w[...], r11b[...],
                    r12w[...], r12b[...])
    nv = _ln(nv, ln2g[...], ln2b[...])
    nv = _gated_res(nv, g2w[...], g2b[...], r21w[...], r21b[...],
                    r22w[...], r22b[...])
    no[...] = nv
    r = jnp.maximum(nv, 0.0)
    heads = _mm(r, dw[...]) + db[...]
    col = lax.broadcasted_iota(jnp.int32, heads.shape, 1)
    ho[...] = jnp.where(col >= 128, jax.nn.sigmoid(heads), heads)


_NC, _NS, _L = 2, 16, 16
_NW = _NC * _NS
_G = 128


def _sc_mesh():
    return plsc.VectorSubcoreMesh(core_axis_name="c", subcore_axis_name="s",
                                  num_cores=_NC, num_subcores=_NS)


def _gather_rows(table, idx2d):
    b_per_w = N // _NW
    n_g = b_per_w // _G

    @functools.partial(
        pl.kernel,
        mesh=_sc_mesh(),
        out_type=jax.ShapeDtypeStruct((N, D), jnp.float32),
        scratch_types=[
            pltpu.VMEM((2 * n_g, _G), jnp.int32),
            pltpu.VMEM((_G, D), jnp.float32),
            pltpu.VMEM((_G, D), jnp.float32),
            pltpu.SemaphoreType.DMA,
            pltpu.SemaphoreType.DMA,
        ],
    )
    def k(table_hbm, idx_hbm, out_hbm, idx_v, rows_a, rows_b, sem_a, sem_b):
        wid = lax.axis_index("s") * _NC + lax.axis_index("c")
        base = wid * b_per_w
        half = wid % 2
        bufs = (rows_a, rows_b)
        sems = (sem_a, sem_b)
        pltpu.sync_copy(idx_hbm.at[pl.ds((wid // 2) * 2 * n_g, 2 * n_g)],
                        idx_v)
        pend = pltpu.async_copy(table_hbm.at[idx_v.at[half * n_g]],
                                bufs[0], sems[0])
        for g in range(n_g):
            if g + 1 < n_g:
                nxt = pltpu.async_copy(
                    table_hbm.at[idx_v.at[half * n_g + g + 1]],
                    bufs[(g + 1) % 2], sems[(g + 1) % 2])
            pend.wait()
            pltpu.sync_copy(bufs[g % 2],
                            out_hbm.at[pl.ds(base + g * _G, _G)])
            if g + 1 < n_g:
                pend = nxt

    return k(table, idx2d)


_HALF = N // 2
_NTRASH = 8


def _scatter_add_rows(x, idx3d):
    CC = 128
    n_chunk = D // CC
    rt = N // _NS
    ng = rt // _G

    @functools.partial(
        pl.kernel,
        mesh=_sc_mesh(),
        out_type=jax.ShapeDtypeStruct((N, D), jnp.float32),
        scratch_types=[
            pltpu.VMEM((ng, _G), jnp.int32),
            pltpu.VMEM((_G, CC), jnp.float32),
            pltpu.VMEM((_G, CC), jnp.float32),
            pltpu.VMEM((_G, CC), jnp.float32),
            pltpu.VMEM_SHARED((_HALF + _NTRASH, CC), jnp.float32),
            pltpu.SemaphoreType.DMA,
            pltpu.SemaphoreType.DMA,
        ],
    )
    def k(x_hbm, idx_hbm, out_hbm, idx_v, xa, xb, zbuf, acc, sem_a, sem_b):
        c = lax.axis_index("c")
        s = lax.axis_index("s")
        row0 = s * rt
        orow0 = s * (_HALF // _NS)
        bufs = (xa, xb)
        sems = (sem_a, sem_b)
        pltpu.sync_copy(idx_hbm.at[c, pl.ds(s * ng, ng)], idx_v)

        def zero_body(i, _):
            r = i // (CC // _L)
            cg = (i % (CC // _L)) * _L
            zbuf[r, pl.ds(cg, _L)] = jnp.zeros((_L,), jnp.float32)
            return 0
        lax.fori_loop(0, _G * (CC // _L), zero_body, 0)
        for r in range(_HALF // _NS // _G):
            pltpu.sync_copy(zbuf, acc.at[pl.ds(orow0 + r * _G, _G)])
        plsc.subcore_barrier()
        for q in range(n_chunk):
            col0 = q * CC
            pend = pltpu.async_copy(
                x_hbm.at[pl.ds(row0, _G), pl.ds(col0, CC)], bufs[0], sems[0])
            for h in range(ng):
                if h + 1 < ng:
                    nxt = pltpu.async_copy(
                        x_hbm.at[pl.ds(row0 + (h + 1) * _G, _G),
                                 pl.ds(col0, CC)],
                        bufs[(h + 1) % 2], sems[(h + 1) % 2])
                pend.wait()
                pltpu.sync_copy(bufs[h % 2], acc.at[idx_v.at[h]], add=True)
                if h + 1 < ng:
                    pend = nxt
            plsc.subcore_barrier()
            pltpu.sync_copy(
                acc.at[pl.ds(orow0, _HALF // _NS)],
                out_hbm.at[pl.ds(c * _HALF + orow0, _HALF // _NS),
                           pl.ds(col0, CC)])
            if q + 1 < n_chunk:
                for r in range(_HALF // _NS // _G):
                    pltpu.sync_copy(zbuf, acc.at[pl.ds(orow0 + r * _G, _G)])
            plsc.subcore_barrier()

    return k(x, idx3d)


def kernel(net, inp, corr, params, ii, jj, ix, jx):
    p = params
    net2 = net.reshape(N, D)
    inp2 = inp.reshape(N, D)
    corr2 = corr.reshape(N, CD)

    def b2(name):
        return p[name].reshape(1, -1)

    def split_idx(idx):
        trash = _HALF + (jnp.arange(N, dtype=jnp.int32) % _NTRASH)
        lo = jnp.where(idx < _HALF, idx, trash)
        hi = jnp.where(idx >= _HALF, idx - _HALF, trash)
        return jnp.stack([lo, hi]).reshape(2, N // _G, _G)

    idx_kk3 = split_idx(jnp.clip(ii.astype(jnp.int32), 0, N - 1))
    idx_ij3 = split_idx(jnp.clip(
        ii.astype(jnp.int32) * 12345 + jj.astype(jnp.int32), 0, N - 1))
    ix_s = jnp.maximum(ix.astype(jnp.int32), 0).reshape(N // _G, _G)
    jx_s = jnp.maximum(jx.astype(jnp.int32), 0).reshape(N // _G, _G)

    net1 = _stage_call(
        _k1_body, [D], [net2, inp2, corr2],
        [p["corr1_w"], b2("corr1_b"), p["corr2_w"], b2("corr2_b"),
         b2("corr_ln_g"), b2("corr_ln_b"), p["corr3_w"], b2("corr3_b"),
         b2("norm_g"), b2("norm_b")])

    g1 = _gather_rows(net1, ix_s)
    net2_ = _stage_call(
        _k2_body, [D], [net1, g1],
        [p["c1a_w"], b2("c1a_b"), p["c1b_w"], b2("c1b_b")])

    g2 = _gather_rows(net2_, jx_s)
    net3, e_kk, fx_kk = _stage_call(
        _k23_body, [D, D, D], [net2_, g2],
        [p["c2a_w"], b2("c2a_b"), p["c2b_w"], b2("c2b_b"),
         p["aggkk_g_w"], b2("aggkk_g_b"), p["aggkk_f_w"], b2("aggkk_f_b")])
    den_kk = _scatter_add_rows(e_kk, idx_kk3)
    c_kk = _stage_call(_k4_body, [D], [e_kk, fx_kk, den_kk], [])
    y_kk = _scatter_add_rows(c_kk, idx_kk3)

    net4, e_ij, fx_ij = _stage_call(
        _k5a_body, [D, D, D], [net3, y_kk],
        [p["aggkk_h_w"], b2("aggkk_h_b"),
         p["aggij_g_w"], b2("aggij_g_b"), p["aggij_f_w"], b2("aggij_f_b")])
    den_ij = _scatter_add_rows(e_ij, idx_ij3)
    c_ij = _stage_call(_k4_body, [D], [e_ij, fx_ij, den_ij], [])
    y_ij = _scatter_add_rows(c_ij, idx_ij3)

    zpad = jnp.zeros((D, 126), jnp.float32)
    dw_pad = jnp.concatenate([p["d_w"], zpad, p["w_w"], zpad], axis=1)
    zb = jnp.zeros((1, 126), jnp.float32)
    db_pad = jnp.concatenate([p["d_b"][None, :], zb, p["w_b"][None, :], zb],
                             axis=1)

    net5, heads = _stage_call(
        _k5b_body, [D, 256], [net4, y_ij],
        [p["aggij_h_w"], b2("aggij_h_b"),
         b2("gru_ln1_g"), b2("gru_ln1_b"),
         p["gr1_gate_w"], b2("gr1_gate_b"), p["gr1_res1_w"], b2("gr1_res1_b"),
         p["gr1_res2_w"], b2("gr1_res2_b"),
         b2("gru_ln2_g"), b2("gru_ln2_b"),
         p["gr2_gate_w"], b2("gr2_gate_b"), p["gr2_res1_w"], b2("gr2_res1_b"),
         p["gr2_res2_w"], b2("gr2_res2_b"), dw_pad, db_pad])

    d_out = heads[:, 0:2].reshape(1, N, 2)
    w_out = heads[:, 128:130].reshape(1, N, 2)
    return net5.reshape(1, N, D), d_out, w_out


# speedup vs baseline: 1.0278x; 1.0278x over previous
import functools

import jax
import jax.numpy as jnp
from jax import lax
from jax.experimental import pallas as pl
from jax.experimental.pallas import tpu as pltpu
from jax.experimental.pallas import tpu_sc as plsc

N = 16384
D = 384
CD = 882
ROWS = 1024
EPS_LN = 1e-3


def _ln(x, g, b):
    mu = jnp.mean(x, axis=-1, keepdims=True)
    var = jnp.mean((x - mu) ** 2, axis=-1, keepdims=True)
    return (x - mu) / jnp.sqrt(var + EPS_LN) * g + b


def _row_spec(cols):
    return pl.BlockSpec((ROWS, cols), lambda i: (i, 0))


def _full_spec(shape):
    nd = len(shape)
    return pl.BlockSpec(shape, lambda i: (0,) * nd)


def _stage_call(body, outs_cols, row_args, full_args):
    grid = N // ROWS
    in_specs = [_row_spec(a.shape[-1]) for a in row_args] + [
        _full_spec(a.shape) for a in full_args
    ]
    out_specs = [_row_spec(c) for c in outs_cols]
    out_shape = [jax.ShapeDtypeStruct((N, c), jnp.float32)
                 for c in outs_cols]
    if len(outs_cols) == 1:
        out_specs, out_shape = out_specs[0], out_shape[0]
    return pl.pallas_call(
        body,
        grid=(grid,),
        in_specs=in_specs,
        out_specs=out_specs,
        out_shape=out_shape,
    )(*row_args, *full_args)


def _mm(x, w):
    return jnp.dot(x, w, preferred_element_type=jnp.float32)


def _k1_body(net, inp, corr, w1, b1, w2, b2, lng, lnb, w3, b3, ng, nb, out):
    h = jnp.maximum(_mm(corr[...], w1[...]) + b1[...], 0.0)
    h = _mm(h, w2[...]) + b2[...]
    h = jnp.maximum(_ln(h, lng[...], lnb[...]), 0.0)
    h = _mm(h, w3[...]) + b3[...]
    out[...] = _ln(net[...] + inp[...] + h, ng[...], nb[...])


def _k2_body(x, g, wa, ba, wb, bb, out):
    t = jnp.maximum(_mm(g[...], wa[...]) + ba[...], 0.0)
    out[...] = x[...] + _mm(t, wb[...]) + bb[...]


def _k23_body(x, g, wa, ba, wb, bb, gw, gb, fw, fb, no, eo, fo):
    t = jnp.maximum(_mm(g[...], wa[...]) + ba[...], 0.0)
    nv = x[...] + _mm(t, wb[...]) + bb[...]
    no[...] = nv
    eo[...] = jnp.exp(_mm(nv, gw[...]) + gb[...])
    fo[...] = _mm(nv, fw[...]) + fb[...]


def _k4_body(e, fx, den, out):
    out[...] = fx[...] * (e[...] / jnp.maximum(den[...], 1e-6))


def _k5a_body(x, y, hw, hb, gw, gb, fw, fb, no, eo, fo):
    nv = x[...] + _mm(y[...], hw[...]) + hb[...]
    no[...] = nv
    eo[...] = jnp.exp(_mm(nv, gw[...]) + gb[...])
    fo[...] = _mm(nv, fw[...]) + fb[...]


def _gated_res(x, gw, gb, r1w, r1b, r2w, r2b):
    gate = jax.nn.sigmoid(_mm(x, gw) + gb)
    r = jnp.maximum(_mm(x, r1w) + r1b, 0.0)
    r = _mm(r, r2w) + r2b
    return x * gate + r


def _k5b_body(x, y, hw, hb, ln1g, ln1b, g1w, g1b, r11w, r11b, r12w, r12b,
              ln2g, ln2b, g2w, g2b, r21w, r21b, r22w, r22b, dw, db, no, ho):
    nv = x[...] + _mm(y[...], hw[...]) + hb[...]
    nv = _ln(nv, ln1g[...], ln1b[...])
    nv = _gated_res(nv, g1w[...], g1b[...], r11w[...], r11b[...],
                    r12w[...], r12b[...])
    nv = _ln(nv, ln2g[...], ln2b[...])
    nv = _gated_res(nv, g2w[...], g2b[...], r21w[...], r21b[...],
                    r22w[...], r22b[...])
    no[...] = nv
    r = jnp.maximum(nv, 0.0)
    heads = _mm(r, dw[...]) + db[...]
    col = lax.broadcasted_iota(jnp.int32, heads.shape, 1)
    ho[...] = jnp.where(col >= 128, jax.nn.sigmoid(heads), heads)


_NC, _NS, _L = 2, 16, 16
_NW = _NC * _NS
_G = 128


def _sc_mesh():
    return plsc.VectorSubcoreMesh(core_axis_name="c", subcore_axis_name="s",
                                  num_cores=_NC, num_subcores=_NS)


def _gather_rows(table, idx2d):
    b_per_w = N // _NW
    n_g = b_per_w // _G

    @functools.partial(
        pl.kernel,
        mesh=_sc_mesh(),
        out_type=jax.ShapeDtypeStruct((N, D), jnp.float32),
        scratch_types=[
            pltpu.VMEM((2 * n_g, _G), jnp.int32),
            pltpu.VMEM((_G, D), jnp.float32),
            pltpu.VMEM((_G, D), jnp.float32),
            pltpu.SemaphoreType.DMA,
            pltpu.SemaphoreType.DMA,
        ],
    )
    def k(table_hbm, idx_hbm, out_hbm, idx_v, rows_a, rows_b, sem_a, sem_b):
        wid = lax.axis_index("s") * _NC + lax.axis_index("c")
        base = wid * b_per_w
        half = wid % 2
        bufs = (rows_a, rows_b)
        sems = (sem_a, sem_b)
        pltpu.sync_copy(idx_hbm.at[pl.ds((wid // 2) * 2 * n_g, 2 * n_g)],
                        idx_v)
        pend = pltpu.async_copy(table_hbm.at[idx_v.at[half * n_g]],
                                bufs[0], sems[0])
        for g in range(n_g):
            if g + 1 < n_g:
                nxt = pltpu.async_copy(
                    table_hbm.at[idx_v.at[half * n_g + g + 1]],
                    bufs[(g + 1) % 2], sems[(g + 1) % 2])
            pend.wait()
            pltpu.sync_copy(bufs[g % 2],
                            out_hbm.at[pl.ds(base + g * _G, _G)])
            if g + 1 < n_g:
                pend = nxt

    return k(table, idx2d)


_HALF = N // 2
_NTRASH = 8


def _scatter_add_rows(x, idx3d):
    CC = 128
    n_chunk = D // CC
    rt = N // _NS
    ng = rt // _G

    @functools.partial(
        pl.kernel,
        mesh=_sc_mesh(),
        out_type=jax.ShapeDtypeStruct((N, D), jnp.float32),
        scratch_types=[
            pltpu.VMEM((ng, _G), jnp.int32),
            pltpu.VMEM((_G, CC), jnp.float32),
            pltpu.VMEM((_G, CC), jnp.float32),
            pltpu.VMEM((_G, CC), jnp.float32),
            pltpu.VMEM_SHARED((_HALF + _NTRASH, CC), jnp.float32),
            pltpu.SemaphoreType.DMA,
            pltpu.SemaphoreType.DMA,
        ],
    )
    def k(x_hbm, idx_hbm, out_hbm, idx_v, xa, xb, zbuf, acc, sem_a, sem_b):
        c = lax.axis_index("c")
        s = lax.axis_index("s")
        row0 = s * rt
        orow0 = s * (_HALF // _NS)
        bufs = (xa, xb)
        sems = (sem_a, sem_b)
        pltpu.sync_copy(idx_hbm.at[c, pl.ds(s * ng, ng)], idx_v)

        def zero_body(i, _):
            r = i // (CC // _L)
            cg = (i % (CC // _L)) * _L
            zbuf[r, pl.ds(cg, _L)] = jnp.zeros((_L,), jnp.float32)
            return 0
        lax.fori_loop(0, _G * (CC // _L), zero_body, 0)
        for q in range(n_chunk):
            col0 = q * CC
            pend = pltpu.async_copy(
                x_hbm.at[pl.ds(row0, _G), pl.ds(col0, CC)], bufs[0], sems[0])
            for r in range(_HALF // _NS // _G):
                pltpu.sync_copy(zbuf, acc.at[pl.ds(orow0 + r * _G, _G)])
            plsc.subcore_barrier()
            for h in range(ng):
                if h + 1 < ng:
                    nxt = pltpu.async_copy(
                        x_hbm.at[pl.ds(row0 + (h + 1) * _G, _G),
                                 pl.ds(col0, CC)],
                        bufs[(h + 1) % 2], sems[(h + 1) % 2])
                pend.wait()
                pltpu.sync_copy(bufs[h % 2], acc.at[idx_v.at[h]], add=True)
                if h + 1 < ng:
                    pend = nxt
            plsc.subcore_barrier()
            pltpu.sync_copy(
                acc.at[pl.ds(orow0, _HALF // _NS)],
                out_hbm.at[pl.ds(c * _HALF + orow0, _HALF // _NS),
                           pl.ds(col0, CC)])
            plsc.subcore_barrier()

    return k(x, idx3d)


def kernel(net, inp, corr, params, ii, jj, ix, jx):
    p = params
    net2 = net.reshape(N, D)
    inp2 = inp.reshape(N, D)
    corr2 = corr.reshape(N, CD)

    def b2(name):
        return p[name].reshape(1, -1)

    def split_idx(idx):
        trash = _HALF + (jnp.arange(N, dtype=jnp.int32) % _NTRASH)
        lo = jnp.where(idx < _HALF, idx, trash)
        hi = jnp.where(idx >= _HALF, idx - _HALF, trash)
        return jnp.stack([lo, hi]).reshape(2, N // _G, _G)

    idx_kk3 = split_idx(jnp.clip(ii.astype(jnp.int32), 0, N - 1))
    idx_ij3 = split_idx(jnp.clip(
        ii.astype(jnp.int32) * 12345 + jj.astype(jnp.int32), 0, N - 1))
    ix_s = jnp.maximum(ix.astype(jnp.int32), 0).reshape(N // _G, _G)
    jx_s = jnp.maximum(jx.astype(jnp.int32), 0).reshape(N // _G, _G)

    net1 = _stage_call(
        _k1_body, [D], [net2, inp2, corr2],
        [p["corr1_w"], b2("corr1_b"), p["corr2_w"], b2("corr2_b"),
         b2("corr_ln_g"), b2("corr_ln_b"), p["corr3_w"], b2("corr3_b"),
         b2("norm_g"), b2("norm_b")])

    g1 = _gather_rows(net1, ix_s)
    net2_ = _stage_call(
        _k2_body, [D], [net1, g1],
        [p["c1a_w"], b2("c1a_b"), p["c1b_w"], b2("c1b_b")])

    g2 = _gather_rows(net2_, jx_s)
    net3, e_kk, fx_kk = _stage_call(
        _k23_body, [D, D, D], [net2_, g2],
        [p["c2a_w"], b2("c2a_b"), p["c2b_w"], b2("c2b_b"),
         p["aggkk_g_w"], b2("aggkk_g_b"), p["aggkk_f_w"], b2("aggkk_f_b")])
    den_kk = _scatter_add_rows(e_kk, idx_kk3)
    c_kk = _stage_call(_k4_body, [D], [e_kk, fx_kk, den_kk], [])
    y_kk = _scatter_add_rows(c_kk, idx_kk3)

    net4, e_ij, fx_ij = _stage_call(
        _k5a_body, [D, D, D], [net3, y_kk],
        [p["aggkk_h_w"], b2("aggkk_h_b"),
         p["aggij_g_w"], b2("aggij_g_b"), p["aggij_f_w"], b2("aggij_f_b")])
    den_ij = _scatter_add_rows(e_ij, idx_ij3)
    c_ij = _stage_call(_k4_body, [D], [e_ij, fx_ij, den_ij], [])
    y_ij = _scatter_add_rows(c_ij, idx_ij3)

    zpad = jnp.zeros((D, 126), jnp.float32)
    dw_pad = jnp.concatenate([p["d_w"], zpad, p["w_w"], zpad], axis=1)
    zb = jnp.zeros((1, 126), jnp.float32)
    db_pad = jnp.concatenate([p["d_b"][None, :], zb, p["w_b"][None, :], zb],
                             axis=1)

    net5, heads = _stage_call(
        _k5b_body, [D, 256], [net4, y_ij],
        [p["aggij_h_w"], b2("aggij_h_b"),
         b2("gru_ln1_g"), b2("gru_ln1_b"),
         p["gr1_gate_w"], b2("gr1_gate_b"), p["gr1_res1_w"], b2("gr1_res1_b"),
         p["gr1_res2_w"], b2("gr1_res2_b"),
         b2("gru_ln2_g"), b2("gru_ln2_b"),
         p["gr2_gate_w"], b2("gr2_gate_b"), p["gr2_res1_w"], b2("gr2_res1_b"),
         p["gr2_res2_w"], b2("gr2_res2_b"), dw_pad, db_pad])

    d_out = heads[:, 0:2].reshape(1, N, 2)
    w_out = heads[:, 128:130].reshape(1, N, 2)
    return net5.reshape(1, N, D), d_out, w_out
